# initial kernel scaffold (unmeasured)
import jax
import jax.numpy as jnp
from jax import lax
from jax.experimental import pallas as pl
from jax.experimental.pallas import tpu as pltpu

K = 4096
M = 4096
N = 8192
N_HALF = N // 2
M_HALF = M // 2

_BM, _BN, _BK = 1024, 1024, 2048
_NCHUNK = 4
_CM = M_HALF // _NCHUNK


def _matmul(x_shard, dy_half):

    def body(x_ref, dy_ref, o_ref):
        @pl.when(pl.program_id(2) == 0)
        def _():
            o_ref[...] = jnp.zeros_like(o_ref)

        o_ref[...] += lax.dot_general(
            x_ref[...],
            dy_ref[...],
            dimension_numbers=(((0,), (0,)), ((), ())),
            preferred_element_type=jnp.float32,
        )

    return pl.pallas_call(
        body,
        grid=(M // _BM, N_HALF // _BN, K // _BK),
        in_specs=[
            pl.BlockSpec((_BK, _BM), lambda i, j, k: (k, i)),
            pl.BlockSpec((_BK, _BN), lambda i, j, k: (k, j)),
        ],
        out_specs=pl.BlockSpec((_BM, _BN), lambda i, j, k: (i, j)),
        out_shape=jax.ShapeDtypeStruct((M, N_HALF), jnp.float32),
        compiler_params=pltpu.CompilerParams(
            dimension_semantics=("parallel", "parallel", "arbitrary"),
        ),
    )(x_shard, dy_half)


def _comm(partial):

    def body(
        partial_ref,
        out_ref,
        recvx_ref,
        a_ref,
        b_ref,
        q_ref,
        copy_sems,
        xchg_send_sem,
        xchg_recv_sem,
        q_send_sems,
        q_recv_sems,
    ):
        my_x = lax.axis_index("x")
        my_y = lax.axis_index("y")
        other_x = 1 - my_x
        other_y = 1 - my_y

        barrier = pltpu.get_barrier_semaphore()
        pl.semaphore_signal(
            barrier, inc=1, device_id=(other_x, my_y),
            device_id_type=pl.DeviceIdType.MESH,
        )
        pl.semaphore_signal(
            barrier, inc=1, device_id=(my_x, other_y),
            device_id_type=pl.DeviceIdType.MESH,
        )
        pl.semaphore_wait(barrier, 2)

        xchg = pltpu.make_async_remote_copy(
            src_ref=partial_ref.at[pl.ds(other_x * M_HALF, M_HALF), :],
            dst_ref=recvx_ref,
            send_sem=xchg_send_sem,
            recv_sem=xchg_recv_sem,
            device_id=(other_x, my_y),
            device_id_type=pl.DeviceIdType.MESH,
        )
        xchg.start()
        xchg.wait()

        for c in range(_NCHUNK):
            row0 = c * _CM
            ca = pltpu.make_async_copy(
                partial_ref.at[pl.ds(my_x * M_HALF + row0, _CM), :],
                a_ref,
                copy_sems.at[0],
            )
            cb = pltpu.make_async_copy(
                recvx_ref.at[pl.ds(row0, _CM), :], b_ref, copy_sems.at[1]
            )
            ca.start()
            cb.start()
            ca.wait()
            cb.wait()
            q_ref[...] = a_ref[...] + b_ref[...]

            cq = pltpu.make_async_copy(
                q_ref,
                out_ref.at[pl.ds(row0, _CM), pl.ds(my_y * N_HALF, N_HALF)],
                copy_sems.at[2],
            )
            cq.start()
            qr = pltpu.make_async_remote_copy(
                src_ref=q_ref,
                dst_ref=out_ref.at[
                    pl.ds(row0, _CM), pl.ds(my_y * N_HALF, N_HALF)
                ],
                send_sem=q_send_sems.at[c],
                recv_sem=q_recv_sems.at[c],
                device_id=(my_x, other_y),
                device_id_type=pl.DeviceIdType.MESH,
            )
            qr.start()
            cq.wait()
            qr.wait()

    return pl.pallas_call(
        body,
        in_specs=[pl.BlockSpec(memory_space=pl.ANY)],
        out_specs=pl.BlockSpec(memory_space=pl.ANY),
        out_shape=jax.ShapeDtypeStruct((M_HALF, N), jnp.float32),
        scratch_shapes=[
            pltpu.MemorySpace.HBM((M_HALF, N_HALF), jnp.float32),
            pltpu.VMEM((_CM, N_HALF), jnp.float32),
            pltpu.VMEM((_CM, N_HALF), jnp.float32),
            pltpu.VMEM((_CM, N_HALF), jnp.float32),
            pltpu.SemaphoreType.DMA((3,)),
            pltpu.SemaphoreType.DMA,
            pltpu.SemaphoreType.DMA,
            pltpu.SemaphoreType.DMA((_NCHUNK,)),
            pltpu.SemaphoreType.DMA((_NCHUNK,)),
        ],
        compiler_params=pltpu.CompilerParams(collective_id=0),
    )(partial)


def kernel(x, dy):
    my_y = lax.axis_index("y")
    dy_half = lax.dynamic_slice(dy, (0, my_y * N_HALF), (K, N_HALF))
    partial = _matmul(x, dy_half)
    return _comm(partial)


# baseline (device time: 1047369 ns/iter reference)
import jax
import jax.numpy as jnp
from jax import lax
from jax.experimental import pallas as pl
from jax.experimental.pallas import tpu as pltpu

K = 4096
M = 4096
N = 8192
N_HALF = N // 2
M_HALF = M // 2

_BM, _BN, _BK = 1024, 1024, 2048
_NCHUNK = 4
_CM = M_HALF // _NCHUNK


def _matmul(x_shard, dy_half):

    def body(x_ref, dy_ref, o_ref):
        @pl.when(pl.program_id(2) == 0)
        def _():
            o_ref[...] = jnp.zeros_like(o_ref)

        o_ref[...] += lax.dot_general(
            x_ref[...],
            dy_ref[...],
            dimension_numbers=(((0,), (0,)), ((), ())),
            preferred_element_type=jnp.float32,
        )

    return pl.pallas_call(
        body,
        grid=(M // _BM, N_HALF // _BN, K // _BK),
        in_specs=[
            pl.BlockSpec((_BK, _BM), lambda i, j, k: (k, i)),
            pl.BlockSpec((_BK, _BN), lambda i, j, k: (k, j)),
        ],
        out_specs=pl.BlockSpec((_BM, _BN), lambda i, j, k: (i, j)),
        out_shape=jax.ShapeDtypeStruct((M, N_HALF), jnp.float32),
        compiler_params=pltpu.CompilerParams(
            dimension_semantics=("parallel", "parallel", "arbitrary"),
            vmem_limit_bytes=64 * 1024 * 1024,
        ),
    )(x_shard, dy_half)


def _comm(partial):

    def body(
        partial_ref,
        out_ref,
        recvx_ref,

        a_ref,
        b_ref,
        q_ref,
        copy_sems,
        xchg_send_sem,
        xchg_recv_sem,
        q_send_sems,
        q_recv_sems,
    ):
        my_x = lax.axis_index("x")
        my_y = lax.axis_index("y")
        other_x = 1 - my_x
        other_y = 1 - my_y

        barrier = pltpu.get_barrier_semaphore()
        pl.semaphore_signal(
            barrier, inc=1, device_id=(other_x, my_y),
            device_id_type=pl.DeviceIdType.MESH,
        )
        pl.semaphore_signal(
            barrier, inc=1, device_id=(my_x, other_y),
            device_id_type=pl.DeviceIdType.MESH,
        )
        pl.semaphore_wait(barrier, 2)

        xchg = pltpu.make_async_remote_copy(
            src_ref=partial_ref.at[pl.ds(other_x * M_HALF, M_HALF), :],
            dst_ref=recvx_ref,
            send_sem=xchg_send_sem,
            recv_sem=xchg_recv_sem,
            device_id=(other_x, my_y),
            device_id_type=pl.DeviceIdType.MESH,
        )
        xchg.start()
        xchg.wait()

        for c in range(_NCHUNK):
            row0 = c * _CM
            ca = pltpu.make_async_copy(
                partial_ref.at[pl.ds(my_x * M_HALF + row0, _CM), :],
                a_ref,
                copy_sems.at[0],
            )
            cb = pltpu.make_async_copy(
                recvx_ref.at[pl.ds(row0, _CM), :], b_ref, copy_sems.at[1]
            )
            ca.start()
            cb.start()
            ca.wait()
            cb.wait()
            q_ref[...] = a_ref[...] + b_ref[...]

            cq = pltpu.make_async_copy(
                q_ref,
                out_ref.at[pl.ds(row0, _CM), pl.ds(my_y * N_HALF, N_HALF)],
                copy_sems.at[2],
            )
            cq.start()
            qr = pltpu.make_async_remote_copy(
                src_ref=q_ref,
                dst_ref=out_ref.at[
                    pl.ds(row0, _CM), pl.ds(my_y * N_HALF, N_HALF)
                ],
                send_sem=q_send_sems.at[c],
                recv_sem=q_recv_sems.at[c],
                device_id=(my_x, other_y),
                device_id_type=pl.DeviceIdType.MESH,
            )
            qr.start()
            cq.wait()
            qr.wait()

    out, _ = pl.pallas_call(
        body,
        in_specs=[pl.BlockSpec(memory_space=pl.ANY)],
        out_specs=[
            pl.BlockSpec(memory_space=pl.ANY),
            pl.BlockSpec(memory_space=pl.ANY),
        ],
        out_shape=[
            jax.ShapeDtypeStruct((M_HALF, N), jnp.float32),
            jax.ShapeDtypeStruct((M_HALF, N_HALF), jnp.float32),
        ],
        scratch_shapes=[
            pltpu.VMEM((_CM, N_HALF), jnp.float32),
            pltpu.VMEM((_CM, N_HALF), jnp.float32),
            pltpu.VMEM((_CM, N_HALF), jnp.float32),
            pltpu.SemaphoreType.DMA((3,)),
            pltpu.SemaphoreType.DMA,
            pltpu.SemaphoreType.DMA,
            pltpu.SemaphoreType.DMA((_NCHUNK,)),
            pltpu.SemaphoreType.DMA((_NCHUNK,)),
        ],
        compiler_params=pltpu.CompilerParams(collective_id=0),
    )(partial)
    return out


def kernel(x, dy):
    my_y = lax.axis_index("y")
    dy_half = lax.dynamic_slice(dy, (0, my_y * N_HALF), (K, N_HALF))
    partial = _matmul(x, dy_half)
    return _comm(partial)


# device time: 547392 ns/iter; 1.9134x vs baseline; 1.9134x over previous
import jax
import jax.numpy as jnp
from jax import lax
from jax.experimental import pallas as pl
from jax.experimental.pallas import tpu as pltpu

K = 4096
M = 4096
N = 8192
N_HALF = N // 2
M_HALF = M // 2

_BM, _BN, _BK = 1024, 1024, 1024
_NI = M // _BM
_NJ = N_HALF // _BN
_NK = K // _BK
_NOWN = _NI // 2


def _fused(scalars, x_shard, dy_shard):
    def body(
        s_ref,
        x_ref,
        dy_ref,
        out_ref,
        recvx_ref,
        stage_ref,
        acc_ref,
        b_ref,
        q_ref,
        copy_sems,
        sx_send,
        sx_recv,
        sy_send,
        sy_recv,
    ):
        i = pl.program_id(0)
        j = pl.program_id(1)
        k = pl.program_id(2)
        my_x = s_ref[0]
        my_y = s_ref[1]
        other_x = 1 - my_x
        other_y = 1 - my_y

        @pl.when((i == 0) & (j == 0) & (k == 0))
        def _():
            barrier = pltpu.get_barrier_semaphore()
            pl.semaphore_signal(
                barrier, inc=1, device_id=(other_x, my_y),
                device_id_type=pl.DeviceIdType.MESH,
            )
            pl.semaphore_signal(
                barrier, inc=1, device_id=(my_x, other_y),
                device_id_type=pl.DeviceIdType.MESH,
            )
            pl.semaphore_wait(barrier, 2)

        @pl.when(k == 0)
        def _():
            acc_ref[...] = jnp.zeros_like(acc_ref)

        acc_ref[...] += lax.dot_general(
            x_ref[...],
            dy_ref[...],
            dimension_numbers=(((0,), (0,)), ((), ())),
            preferred_element_type=jnp.float32,
        )

        @pl.when(k == _NK - 1)
        def _finalize():
            rows = pl.ds(i * _BM, _BM)
            rows_own = pl.ds((i - _NOWN) * _BM, _BM)
            cols = pl.ds(j * _BN, _BN)

            @pl.when(i < _NOWN)
            def _send_to_x_neighbor():
                st = pltpu.make_async_copy(
                    acc_ref, stage_ref.at[rows, cols], copy_sems.at[0]
                )
                st.start()
                st.wait()
                pltpu.make_async_remote_copy(
                    src_ref=stage_ref.at[rows, cols],
                    dst_ref=recvx_ref.at[rows, cols],
                    send_sem=sx_send.at[i, j],
                    recv_sem=sx_recv.at[i, j],
                    device_id=(other_x, my_y),
                    device_id_type=pl.DeviceIdType.MESH,
                ).start()

            @pl.when(i >= _NOWN)
            def _reduce_and_send_y():
                pltpu.make_async_remote_copy(
                    src_ref=stage_ref.at[rows_own, cols],
                    dst_ref=recvx_ref.at[rows_own, cols],
                    send_sem=sx_send.at[i - _NOWN, j],
                    recv_sem=sx_recv.at[i - _NOWN, j],
                    device_id=(other_x, my_y),
                    device_id_type=pl.DeviceIdType.MESH,
                ).wait_recv()
                ld = pltpu.make_async_copy(
                    recvx_ref.at[rows_own, cols], b_ref, copy_sems.at[1]
                )
                ld.start()
                ld.wait()
                q_ref[...] = acc_ref[...] + b_ref[...]
                out_cols = pl.ds(my_y * N_HALF + j * _BN, _BN)
                stq = pltpu.make_async_copy(
                    q_ref, out_ref.at[rows_own, out_cols], copy_sems.at[2]
                )
                stq.start()
                stq.wait()
                pltpu.make_async_remote_copy(
                    src_ref=out_ref.at[rows_own, out_cols],
                    dst_ref=out_ref.at[rows_own, out_cols],
                    send_sem=sy_send.at[i - _NOWN, j],
                    recv_sem=sy_recv.at[i - _NOWN, j],
                    device_id=(my_x, other_y),
                    device_id_type=pl.DeviceIdType.MESH,
                ).start()

        @pl.when((i == _NI - 1) & (j == _NJ - 1) & (k == _NK - 1))
        def _drain():
            for ii in range(_NOWN):
                for jj in range(_NJ):
                    rs = pl.ds(ii * _BM, _BM)
                    cs = pl.ds(jj * _BN, _BN)
                    ocs = pl.ds(my_y * N_HALF + jj * _BN, _BN)
                    rcs = pl.ds(other_y * N_HALF + jj * _BN, _BN)
                    pltpu.make_async_remote_copy(
                        src_ref=stage_ref.at[rs, cs],
                        dst_ref=recvx_ref.at[rs, cs],
                        send_sem=sx_send.at[ii, jj],
                        recv_sem=sx_recv.at[ii, jj],
                        device_id=(other_x, my_y),
                        device_id_type=pl.DeviceIdType.MESH,
                    ).wait_send()
                    pltpu.make_async_remote_copy(
                        src_ref=out_ref.at[rs, ocs],
                        dst_ref=out_ref.at[rs, ocs],
                        send_sem=sy_send.at[ii, jj],
                        recv_sem=sy_recv.at[ii, jj],
                        device_id=(my_x, other_y),
                        device_id_type=pl.DeviceIdType.MESH,
                    ).wait_send()
                    pltpu.make_async_remote_copy(
                        src_ref=out_ref.at[rs, ocs],
                        dst_ref=out_ref.at[rs, rcs],
                        send_sem=sy_send.at[ii, jj],
                        recv_sem=sy_recv.at[ii, jj],
                        device_id=(my_x, other_y),
                        device_id_type=pl.DeviceIdType.MESH,
                    ).wait_recv()

    grid_spec = pltpu.PrefetchScalarGridSpec(
        num_scalar_prefetch=1,
        grid=(_NI, _NJ, _NK),
        in_specs=[
            pl.BlockSpec(
                (_BK, _BM), lambda i, j, k, s: (k, (i + 2 * (1 - s[0])) % 4)
            ),
            pl.BlockSpec((_BK, _BN), lambda i, j, k, s: (k, s[1] * _NJ + j)),
        ],
        out_specs=[
            pl.BlockSpec(memory_space=pl.ANY),
            pl.BlockSpec(memory_space=pl.ANY),
            pl.BlockSpec(memory_space=pl.ANY),
        ],
        scratch_shapes=[
            pltpu.VMEM((_BM, _BN), jnp.float32),
            pltpu.VMEM((_BM, _BN), jnp.float32),
            pltpu.VMEM((_BM, _BN), jnp.float32),
            pltpu.SemaphoreType.DMA((3,)),
            pltpu.SemaphoreType.DMA((_NOWN, _NJ)),
            pltpu.SemaphoreType.DMA((_NOWN, _NJ)),
            pltpu.SemaphoreType.DMA((_NOWN, _NJ)),
            pltpu.SemaphoreType.DMA((_NOWN, _NJ)),
        ],
    )
    out, _, _ = pl.pallas_call(
        body,
        grid_spec=grid_spec,
        out_shape=[
            jax.ShapeDtypeStruct((M_HALF, N), jnp.float32),
            jax.ShapeDtypeStruct((M_HALF, N_HALF), jnp.float32),
            jax.ShapeDtypeStruct((M_HALF, N_HALF), jnp.float32),
        ],
        compiler_params=pltpu.CompilerParams(
            dimension_semantics=("arbitrary", "arbitrary", "arbitrary"),
            vmem_limit_bytes=64 * 1024 * 1024,
            collective_id=0,
        ),
    )(scalars, x_shard, dy_shard)
    return out


def kernel(x, dy):
    my_x = lax.axis_index("x")
    my_y = lax.axis_index("y")
    scalars = jnp.array([my_x, my_y], dtype=jnp.int32)
    return _fused(scalars, x, dy)


# device time: 537356 ns/iter; 1.9491x vs baseline; 1.0187x over previous
import jax
import jax.numpy as jnp
from jax import lax
from jax.experimental import pallas as pl
from jax.experimental.pallas import tpu as pltpu

K = 4096
M = 4096
N = 8192
N_HALF = N // 2
M_HALF = M // 2

_BM, _BN, _BK = 1024, 1024, 2048
_NI = M // _BM
_NJ = N_HALF // _BN
_NK = K // _BK
_NOWN = _NI // 2


def _fused(scalars, x_shard, dy_shard):
    def body(
        s_ref,
        x_ref,
        dy_ref,
        out_ref,
        recvx_ref,
        stage_ref,
        acc_ref,
        b_ref,
        q_ref,
        copy_sems,
        sx_send,
        sx_recv,
        sy_send,
        sy_recv,
    ):
        i = pl.program_id(0)
        j = pl.program_id(1)
        k = pl.program_id(2)
        my_x = s_ref[0]
        my_y = s_ref[1]
        other_x = 1 - my_x
        other_y = 1 - my_y

        @pl.when((i == 0) & (j == 0) & (k == 0))
        def _():
            barrier = pltpu.get_barrier_semaphore()
            pl.semaphore_signal(
                barrier, inc=1, device_id=(other_x, my_y),
                device_id_type=pl.DeviceIdType.MESH,
            )
            pl.semaphore_signal(
                barrier, inc=1, device_id=(my_x, other_y),
                device_id_type=pl.DeviceIdType.MESH,
            )
            pl.semaphore_wait(barrier, 2)

        @pl.when(k == 0)
        def _():
            acc_ref[...] = jnp.zeros_like(acc_ref)

        acc_ref[...] += lax.dot_general(
            x_ref[...],
            dy_ref[...],
            dimension_numbers=(((0,), (0,)), ((), ())),
            preferred_element_type=jnp.float32,
        )

        @pl.when(k == _NK - 1)
        def _finalize():
            rows = pl.ds(i * _BM, _BM)
            rows_own = pl.ds((i - _NOWN) * _BM, _BM)
            cols = pl.ds(j * _BN, _BN)

            @pl.when(i < _NOWN)
            def _send_to_x_neighbor():
                st = pltpu.make_async_copy(
                    acc_ref, stage_ref.at[rows, cols], copy_sems.at[0]
                )
                st.start()
                st.wait()
                pltpu.make_async_remote_copy(
                    src_ref=stage_ref.at[rows, cols],
                    dst_ref=recvx_ref.at[rows, cols],
                    send_sem=sx_send.at[i, j],
                    recv_sem=sx_recv.at[i, j],
                    device_id=(other_x, my_y),
                    device_id_type=pl.DeviceIdType.MESH,
                ).start()

            @pl.when(i >= _NOWN)
            def _reduce_and_send_y():
                pltpu.make_async_remote_copy(
                    src_ref=stage_ref.at[rows_own, cols],
                    dst_ref=recvx_ref.at[rows_own, cols],
                    send_sem=sx_send.at[i - _NOWN, j],
                    recv_sem=sx_recv.at[i - _NOWN, j],
                    device_id=(other_x, my_y),
                    device_id_type=pl.DeviceIdType.MESH,
                ).wait_recv()
                ld = pltpu.make_async_copy(
                    recvx_ref.at[rows_own, cols], b_ref, copy_sems.at[1]
                )
                ld.start()
                ld.wait()
                q_ref[...] = acc_ref[...] + b_ref[...]
                out_cols = pl.ds(my_y * N_HALF + j * _BN, _BN)
                stq = pltpu.make_async_copy(
                    q_ref, out_ref.at[rows_own, out_cols], copy_sems.at[2]
                )
                stq.start()
                stq.wait()
                pltpu.make_async_remote_copy(
                    src_ref=out_ref.at[rows_own, out_cols],
                    dst_ref=out_ref.at[rows_own, out_cols],
                    send_sem=sy_send.at[i - _NOWN, j],
                    recv_sem=sy_recv.at[i - _NOWN, j],
                    device_id=(my_x, other_y),
                    device_id_type=pl.DeviceIdType.MESH,
                ).start()

        @pl.when((i == _NI - 1) & (j == _NJ - 1) & (k == _NK - 1))
        def _drain():
            for ii in range(_NOWN):
                for jj in range(_NJ):
                    rs = pl.ds(ii * _BM, _BM)
                    cs = pl.ds(jj * _BN, _BN)
                    ocs = pl.ds(my_y * N_HALF + jj * _BN, _BN)
                    rcs = pl.ds(other_y * N_HALF + jj * _BN, _BN)
                    pltpu.make_async_remote_copy(
                        src_ref=stage_ref.at[rs, cs],
                        dst_ref=recvx_ref.at[rs, cs],
                        send_sem=sx_send.at[ii, jj],
                        recv_sem=sx_recv.at[ii, jj],
                        device_id=(other_x, my_y),
                        device_id_type=pl.DeviceIdType.MESH,
                    ).wait_send()
                    pltpu.make_async_remote_copy(
                        src_ref=out_ref.at[rs, ocs],
                        dst_ref=out_ref.at[rs, ocs],
                        send_sem=sy_send.at[ii, jj],
                        recv_sem=sy_recv.at[ii, jj],
                        device_id=(my_x, other_y),
                        device_id_type=pl.DeviceIdType.MESH,
                    ).wait_send()
                    pltpu.make_async_remote_copy(
                        src_ref=out_ref.at[rs, ocs],
                        dst_ref=out_ref.at[rs, rcs],
                        send_sem=sy_send.at[ii, jj],
                        recv_sem=sy_recv.at[ii, jj],
                        device_id=(my_x, other_y),
                        device_id_type=pl.DeviceIdType.MESH,
                    ).wait_recv()

    grid_spec = pltpu.PrefetchScalarGridSpec(
        num_scalar_prefetch=1,
        grid=(_NI, _NJ, _NK),
        in_specs=[
            pl.BlockSpec(
                (_BK, _BM), lambda i, j, k, s: (k, (i + 2 * (1 - s[0])) % 4)
            ),
            pl.BlockSpec((_BK, _BN), lambda i, j, k, s: (k, s[1] * _NJ + j)),
        ],
        out_specs=[
            pl.BlockSpec(memory_space=pl.ANY),
            pl.BlockSpec(memory_space=pl.ANY),
            pl.BlockSpec(memory_space=pl.ANY),
        ],
        scratch_shapes=[
            pltpu.VMEM((_BM, _BN), jnp.float32),
            pltpu.VMEM((_BM, _BN), jnp.float32),
            pltpu.VMEM((_BM, _BN), jnp.float32),
            pltpu.SemaphoreType.DMA((3,)),
            pltpu.SemaphoreType.DMA((_NOWN, _NJ)),
            pltpu.SemaphoreType.DMA((_NOWN, _NJ)),
            pltpu.SemaphoreType.DMA((_NOWN, _NJ)),
            pltpu.SemaphoreType.DMA((_NOWN, _NJ)),
        ],
    )
    out, _, _ = pl.pallas_call(
        body,
        grid_spec=grid_spec,
        out_shape=[
            jax.ShapeDtypeStruct((M_HALF, N), jnp.float32),
            jax.ShapeDtypeStruct((M_HALF, N_HALF), jnp.float32),
            jax.ShapeDtypeStruct((M_HALF, N_HALF), jnp.float32),
        ],
        compiler_params=pltpu.CompilerParams(
            dimension_semantics=("arbitrary", "arbitrary", "arbitrary"),
            vmem_limit_bytes=64 * 1024 * 1024,
            collective_id=0,
        ),
    )(scalars, x_shard, dy_shard)
    return out


def kernel(x, dy):
    my_x = lax.axis_index("x")
    my_y = lax.axis_index("y")
    scalars = jnp.array([my_x, my_y], dtype=jnp.int32)
    return _fused(scalars, x, dy)
